# baseline (speedup 1.0000x reference)
import functools
import jax
import jax.numpy as jnp
from jax import lax
from jax.experimental import pallas as pl
from jax.experimental.pallas import tpu as pltpu
from jax.experimental.pallas import tpu_sc as plsc

DEPTH = 1000
ROWS = 81920
NW = 32
ROWS_PER_W = ROWS // NW
CHUNK_ROWS = 128
CHUNK_WORDS = CHUNK_ROWS * DEPTH
NCHUNK = ROWS_PER_W // CHUNK_ROWS


def _sc_body(idx_hbm, out_hbm, idx_v, buf_v, sem):
    wid = lax.axis_index("s") * 2 + lax.axis_index("c")
    base_row = wid * ROWS_PER_W

    pltpu.sync_copy(idx_hbm.at[pl.ds(base_row, ROWS_PER_W)], idx_v)

    def zero_body(i, _):
        buf_v[pl.ds(i * 16, 16)] = jnp.zeros((16,), jnp.float32)
        return 0
    lax.fori_loop(0, CHUNK_WORDS // 16, zero_body, 0)

    lane = lax.iota(jnp.int32, 16)

    def chunk_body(k, _):
        def ones_body(j, _):
            idxs = idx_v[pl.ds(k * CHUNK_ROWS + j * 16, 16)]
            offs = (j * 16 + lane) * DEPTH + idxs
            plsc.store_scatter(buf_v, [offs], jnp.ones((16,), jnp.float32))
            return 0
        lax.fori_loop(0, CHUNK_ROWS // 16, ones_body, 0)

        dst0 = (base_row + k * CHUNK_ROWS) * DEPTH
        pltpu.async_copy(
            buf_v, out_hbm.at[pl.ds(dst0, CHUNK_WORDS)], sem
        ).wait()

        def clear_body(j, _):
            idxs = idx_v[pl.ds(k * CHUNK_ROWS + j * 16, 16)]
            offs = (j * 16 + lane) * DEPTH + idxs
            plsc.store_scatter(buf_v, [offs], jnp.zeros((16,), jnp.float32))
            return 0
        lax.fori_loop(0, CHUNK_ROWS // 16, clear_body, 0)
        return 0

    lax.fori_loop(0, NCHUNK, chunk_body, 0)


def kernel(inputs):
    n, m = inputs.shape
    idx_flat = inputs.reshape(ROWS)
    mesh = plsc.VectorSubcoreMesh(core_axis_name="c", subcore_axis_name="s")
    k = functools.partial(
        pl.kernel,
        mesh=mesh,
        out_type=jax.ShapeDtypeStruct((ROWS * DEPTH,), jnp.float32),
        scratch_types=[
            pltpu.VMEM((ROWS_PER_W,), jnp.int32),
            pltpu.VMEM((CHUNK_WORDS,), jnp.float32),
            pltpu.SemaphoreType.DMA,
        ],
        compiler_params=pltpu.CompilerParams(needs_layout_passes=False),
    )(_sc_body)
    out = k(idx_flat)
    return out.reshape(n, m, DEPTH)


# speedup vs baseline: 7.3332x; 7.3332x over previous
import jax
import jax.numpy as jnp
from jax.experimental import pallas as pl

DEPTH = 1000
R = 1024


def _onehot_block(idx_ref, out_ref):
    idx = idx_ref[...]
    iota = jax.lax.broadcasted_iota(idx.dtype, out_ref.shape, 1)
    out_ref[...] = (idx == iota).astype(jnp.float32)


def kernel(inputs):
    n, m = inputs.shape
    idx_t = inputs.T.reshape(m, 1, n)
    out_t = pl.pallas_call(
        _onehot_block,
        grid=(m, n // R),
        in_specs=[pl.BlockSpec((1, 1, R), lambda j, i: (j, 0, i))],
        out_specs=pl.BlockSpec((1, DEPTH, R), lambda j, i: (j, 0, i)),
        out_shape=jax.ShapeDtypeStruct((m, DEPTH, n), jnp.float32),
    )(idx_t)
    return out_t.transpose(2, 0, 1)
